# baseline (speedup 1.0000x reference)
import functools

import jax
import jax.numpy as jnp
from jax import lax
from jax.experimental import pallas as pl
from jax.experimental.pallas import tpu as pltpu
from jax.experimental.pallas import tpu_sc as plsc

G, N, F, H, B = 8, 1024, 128, 64, 16
GPB = 2
HH = H // 2


def _dot(p, q):
    return jnp.dot(p, q, preferred_element_type=jnp.float32)


def _gnn_body(xs_ref, a_ref, win_ref, bin_ref, w0_ref, b0_ref, w1_ref,
              b1_ref, w2_ref, b2_ref, out_ref):
    a = [a_ref[k] for k in range(GPB)]
    win, bin_ = win_ref[...], bin_ref[...]
    ws = [(w0_ref[...], b0_ref[...]), (w1_ref[...], b1_ref[...])]
    w2, b2 = w2_ref[...], b2_ref[...]

    x = [jnp.maximum(_dot(xs_ref[k], win) + bin_, 0.0) for k in range(GPB)]
    t = x
    for w, b in ws:
        y = [_dot(x[k], w) for k in range(GPB)]
        x = [jnp.maximum(_dot(a[k], y[k]) + b, 0.0) for k in range(GPB)]
    y = [_dot(x[k], w2) for k in range(GPB)]
    z = [_dot(a[k], y[k]) + b2 for k in range(GPB)]
    e = [jnp.exp(z[k]) for k in range(GPB)]
    x = [e[k] / jnp.sum(e[k], axis=-1, keepdims=True) + t[k]
         for k in range(GPB)]
    for k in range(GPB):
        out_ref[k] = x[k].T


_GDN = lax.GatherDimensionNumbers(
    offset_dims=(), collapsed_slice_dims=(0,), start_index_map=(0,))


def _perm(v, idx):
    return lax.gather(v, idx.reshape(16, 1), dimension_numbers=_GDN,
                      slice_sizes=(1,),
                      mode=lax.GatherScatterMode.PROMISE_IN_BOUNDS)


def _allsum(v, lanes):
    for sh in (8, 4, 2, 1):
        v = v + _perm(v, lax.bitwise_xor(lanes, sh))
    return v


def _sc_pool(hidt_hbm, gids_hbm, mask_hbm, out_hbm,
             gid_v, m_v, ht_v, obuf_v, sem):
    info = plsc.get_sparse_core_info()
    nc = info.num_cores
    wid = lax.axis_index("s") * nc + lax.axis_index("c")
    b = wid % B
    hh = wid // B

    pltpu.sync_copy(gids_hbm, gid_v)
    pltpu.sync_copy(mask_hbm.at[b], m_v)

    lanes = lax.iota(jnp.int32, 16)
    gb = _perm(gid_v[...], lanes * 0 + b)
    for j in range(2):
        rows = gb * H + (hh * HH + j * 16) + lanes
        pltpu.async_copy(hidt_hbm.at[rows], ht_v.at[j], sem).wait()

    cvec = jnp.zeros((16,), jnp.float32)
    for i in range(N // 16):
        cvec = cvec + m_v[pl.ds(i * 16, 16)]
    cntv = _allsum(cvec, lanes)
    rcnt = 1.0 / jnp.maximum(cntv, 1.0)

    def body(i, accs):
        mchunk = m_v[pl.ds(i * 16, 16)]
        return tuple(
            accs[j * 16 + k] + mchunk * ht_v[j, k, pl.ds(i * 16, 16)]
            for j in range(2) for k in range(16))
    init = tuple(jnp.zeros((16,), jnp.float32) for _ in range(HH))
    accs = lax.fori_loop(0, N // 16, body, init)

    o0 = jnp.zeros((16,), jnp.float32)
    o1 = jnp.zeros((16,), jnp.float32)
    for h in range(HH):
        s = _allsum(accs[h], lanes) * rcnt
        if h < 16:
            o0 = jnp.where(lanes == h, s, o0)
        else:
            o1 = jnp.where(lanes == (h - 16), s, o1)
    obuf_v[pl.ds(0, 16)] = o0
    obuf_v[pl.ds(16, 16)] = o1
    pltpu.sync_copy(obuf_v, out_hbm.at[b, hh])


@jax.jit
def kernel(cdfg_xs, cdfg_as, W_in, b_in, W0, b0, W1, b1, W2, b2, graph,
           coverpoint, coverpoint_mask):
    del coverpoint
    gids = graph.astype(jnp.int32).reshape(B)
    maskf = coverpoint_mask.astype(jnp.float32)

    hidden_t = pl.pallas_call(
        _gnn_body,
        grid=(G // GPB,),
        in_specs=[
            pl.BlockSpec((GPB, N, F), lambda g: (g, 0, 0)),
            pl.BlockSpec((GPB, N, N), lambda g: (g, 0, 0)),
            pl.BlockSpec((F, H), lambda g: (0, 0)),
            pl.BlockSpec((1, H), lambda g: (0, 0)),
            pl.BlockSpec((H, H), lambda g: (0, 0)),
            pl.BlockSpec((1, H), lambda g: (0, 0)),
            pl.BlockSpec((H, H), lambda g: (0, 0)),
            pl.BlockSpec((1, H), lambda g: (0, 0)),
            pl.BlockSpec((H, H), lambda g: (0, 0)),
            pl.BlockSpec((1, H), lambda g: (0, 0)),
        ],
        out_specs=pl.BlockSpec((GPB, H, N), lambda g: (g, 0, 0)),
        out_shape=jax.ShapeDtypeStruct((G, H, N), jnp.float32),
    )(cdfg_xs, cdfg_as, W_in, b_in.reshape(1, H), W0, b0.reshape(1, H),
      W1, b1.reshape(1, H), W2, b2.reshape(1, H))

    mesh = plsc.VectorSubcoreMesh(core_axis_name="c", subcore_axis_name="s")
    pool = functools.partial(
        pl.kernel, mesh=mesh,
        out_type=jax.ShapeDtypeStruct((B, 2, HH), jnp.float32),
        scratch_types=[
            pltpu.VMEM((B,), jnp.int32),
            pltpu.VMEM((N,), jnp.float32),
            pltpu.VMEM((2, 16, N), jnp.float32),
            pltpu.VMEM((HH,), jnp.float32),
            pltpu.SemaphoreType.DMA,
        ],
    )(_sc_pool)
    out = pool(hidden_t.reshape(G * H, N), gids, maskf)
    return out.reshape(B, H)


# speedup vs baseline: 1.8894x; 1.8894x over previous
import jax
import jax.numpy as jnp
from jax.experimental import pallas as pl

G, N, F, H, B = 8, 1024, 128, 64, 16
GPB = 2


def _dot(p, q):
    return jnp.dot(p, q, preferred_element_type=jnp.float32)


def _gnn_body(xs_ref, a_ref, win_ref, bin_ref, w0_ref, b0_ref, w1_ref,
              b1_ref, w2_ref, b2_ref, gids_ref, mask_ref, out_ref):
    step = pl.program_id(0)
    a = [a_ref[k] for k in range(GPB)]
    win, bin_ = win_ref[...], bin_ref[...]
    ws = [(w0_ref[...], b0_ref[...]), (w1_ref[...], b1_ref[...])]
    w2, b2 = w2_ref[...], b2_ref[...]

    x = [jnp.maximum(_dot(xs_ref[k], win) + bin_, 0.0) for k in range(GPB)]
    t = x
    for w, b in ws:
        y = [_dot(x[k], w) for k in range(GPB)]
        x = [jnp.maximum(_dot(a[k], y[k]) + b, 0.0) for k in range(GPB)]
    y = [_dot(x[k], w2) for k in range(GPB)]
    z = [_dot(a[k], y[k]) + b2 for k in range(GPB)]
    e = [jnp.exp(z[k]) for k in range(GPB)]
    x = [e[k] / jnp.sum(e[k], axis=-1, keepdims=True) + t[k]
         for k in range(GPB)]

    m = mask_ref[...]
    pm = [_dot(m, x[k]) for k in range(GPB)]
    cnt = jnp.maximum(jnp.sum(m, axis=1, keepdims=True), 1.0)

    @pl.when(step == 0)
    def _init():
        out_ref[...] = jnp.zeros_like(out_ref)

    acc = out_ref[...]
    for k in range(GPB):
        sel = gids_ref[...] == (step * GPB + k)
        acc = jnp.where(sel, pm[k] / cnt, acc)
    out_ref[...] = acc


@jax.jit
def kernel(cdfg_xs, cdfg_as, W_in, b_in, W0, b0, W1, b1, W2, b2, graph,
           coverpoint, coverpoint_mask):
    del coverpoint
    gids = graph.astype(jnp.int32).reshape(B, 1)
    maskf = coverpoint_mask.astype(jnp.float32)

    out = pl.pallas_call(
        _gnn_body,
        grid=(G // GPB,),
        in_specs=[
            pl.BlockSpec((GPB, N, F), lambda g: (g, 0, 0)),
            pl.BlockSpec((GPB, N, N), lambda g: (g, 0, 0)),
            pl.BlockSpec((F, H), lambda g: (0, 0)),
            pl.BlockSpec((1, H), lambda g: (0, 0)),
            pl.BlockSpec((H, H), lambda g: (0, 0)),
            pl.BlockSpec((1, H), lambda g: (0, 0)),
            pl.BlockSpec((H, H), lambda g: (0, 0)),
            pl.BlockSpec((1, H), lambda g: (0, 0)),
            pl.BlockSpec((H, H), lambda g: (0, 0)),
            pl.BlockSpec((1, H), lambda g: (0, 0)),
            pl.BlockSpec((B, 1), lambda g: (0, 0)),
            pl.BlockSpec((B, N), lambda g: (0, 0)),
        ],
        out_specs=pl.BlockSpec((B, H), lambda g: (0, 0)),
        out_shape=jax.ShapeDtypeStruct((B, H), jnp.float32),
    )(cdfg_xs, cdfg_as, W_in, b_in.reshape(1, H), W0, b0.reshape(1, H),
      W1, b1.reshape(1, H), W2, b2.reshape(1, H), gids, maskf)
    return out
